# Initial kernel scaffold; baseline (speedup 1.0000x reference)
import jax
import jax.numpy as jnp
from jax.experimental import pallas as pl


def kernel(atom_embed, fragment_embed, index):
    raise NotImplementedError("write your pallas kernel here")


# speedup vs baseline: 8.9746x; 8.9746x over previous
import jax
import jax.numpy as jnp
from jax import lax
from jax.experimental import pallas as pl
from jax.experimental.pallas import tpu as pltpu

B, A, D, F_ = 16, 2048, 256, 128
TEMP = 0.1
EPS = 1e-8


def _dense_body(idx_ref, ae_ref, fe_ref, loss_ref, cnt_ref):
    b = pl.program_id(0)
    idx = idx_ref[0, 0]
    ae = ae_ref[0]
    fe = fe_ref[0]

    frag_ids = lax.broadcasted_iota(jnp.int32, (A, F_), 1)
    onehot = (idx[:, None] == frag_ids).astype(jnp.float32)
    sums = lax.dot_general(onehot, ae, (((0,), (0,)), ((), ())),
                           preferred_element_type=jnp.float32)
    counts = jnp.sum(onehot, axis=0)

    valid = counts > 0.0
    mean = sums / jnp.maximum(counts, 1.0)[:, None]
    mn = jnp.maximum(jnp.sqrt(jnp.sum(mean * mean, axis=1, keepdims=True)), EPS)
    fn = jnp.maximum(jnp.sqrt(jnp.sum(fe * fe, axis=1, keepdims=True)), EPS)
    sims = lax.dot_general(mean / mn, fe / fn, (((1,), (1,)), ((), ())),
                           preferred_element_type=jnp.float32) / TEMP

    eye = (lax.broadcasted_iota(jnp.int32, (F_, F_), 0)
           == lax.broadcasted_iota(jnp.int32, (F_, F_), 1)).astype(jnp.float32)
    pos = jnp.sum(sims * eye, axis=1)
    m = jnp.max(sims, axis=1)
    lse = m + jnp.log(jnp.sum(jnp.exp(sims - m[:, None]), axis=1))
    loss_f = lse - pos

    item_loss = jnp.sum(jnp.where(valid, loss_f, 0.0))
    item_cnt = jnp.sum(valid.astype(jnp.float32))

    @pl.when(b == 0)
    def _():
        loss_ref[...] = jnp.zeros_like(loss_ref)
        cnt_ref[...] = jnp.zeros_like(cnt_ref)

    loss_ref[...] += item_loss.reshape(1, 1)
    cnt_ref[...] += item_cnt.reshape(1, 1)


def kernel(atom_embed, fragment_embed, index):
    loss, cnt = pl.pallas_call(
        _dense_body,
        grid=(B,),
        in_specs=[
            pl.BlockSpec((1, 1, A), lambda b: (b, 0, 0)),
            pl.BlockSpec((1, A, D), lambda b: (b, 0, 0)),
            pl.BlockSpec((1, F_, D), lambda b: (b, 0, 0)),
        ],
        out_specs=[
            pl.BlockSpec((1, 1), lambda b: (0, 0)),
            pl.BlockSpec((1, 1), lambda b: (0, 0)),
        ],
        out_shape=[
            jax.ShapeDtypeStruct((1, 1), jnp.float32),
            jax.ShapeDtypeStruct((1, 1), jnp.float32),
        ],
    )(index.reshape(B, 1, A), atom_embed, fragment_embed)
    total = loss[0, 0]
    c = cnt[0, 0]
    return jnp.where(c > 0, total / c, jnp.float32(0.0))
